# baseline (speedup 1.0000x reference)
import functools

import jax
import jax.numpy as jnp
from jax import lax
from jax.experimental import pallas as pl
from jax.experimental.pallas import tpu as pltpu
from jax.experimental.pallas import tpu_sc as plsc

N_NODES = 10000
F = 16
NPAD = 10112
ROWS_PER_SUB = NPAD // 16
DUMMY = 10008
NCORES = 2
NSUB = 16
NTILES = NCORES * NSUB
CHUNK = 128

_mesh = plsc.VectorSubcoreMesh(core_axis_name="c", subcore_axis_name="s")


NBUF = 2
INNER = 2


def _agg_body(nchunk, g_hbm, init_hbm, src_hbm, dst_hbm, out_hbm,
              sidx, didx, rows, stage, acc, *sems):
    c = lax.axis_index("c")
    s = lax.axis_index("s")
    t = c * NSUB + s
    sl = pl.ds(s * ROWS_PER_SUB, ROWS_PER_SUB)
    pltpu.sync_copy(init_hbm.at[c, sl], stage)
    pltpu.sync_copy(stage, acc.at[sl])
    pltpu.sync_copy(src_hbm.at[t], sidx)
    pltpu.sync_copy(dst_hbm.at[t], didx)
    plsc.subcore_barrier()

    def _gather(j, b):
        return pltpu.make_async_copy(
            g_hbm.at[sidx.at[j]], rows.at[b], sems[b])

    _gather(0, 0).start()

    def _group(gi, carry):
        for k in range(INNER):
            j = gi * INNER + k
            _gather(j, k).wait()
            _gather(j + 1, 1 - k).start()
            pltpu.sync_copy(rows.at[k], acc.at[didx.at[j]], add=True)
        return carry

    lax.fori_loop(0, nchunk // INNER, _group, 0)
    _gather(0, 0).wait()
    plsc.subcore_barrier()
    pltpu.sync_copy(acc.at[sl], out_hbm.at[c, sl])


def _make_agg(nchunk):
    return functools.partial(
        pl.kernel,
        mesh=_mesh,
        compiler_params=pltpu.CompilerParams(use_tc_tiling_on_sc=False),
        out_type=jax.ShapeDtypeStruct((NCORES, NPAD, F), jnp.float32),
        scratch_types=[
            pltpu.VMEM((nchunk + NBUF, CHUNK), jnp.int32),
            pltpu.VMEM((nchunk + NBUF, CHUNK), jnp.int32),
            pltpu.VMEM((NBUF, CHUNK, F), jnp.float32),
            pltpu.VMEM((ROWS_PER_SUB, F), jnp.float32),
            pltpu.VMEM_SHARED((NPAD, F), jnp.float32),
        ] + [pltpu.SemaphoreType.DMA] * NBUF,
    )(functools.partial(_agg_body, nchunk))


def _tc1_body(x_ref, w1_ref, cnt_ref, g1_ref, dis_ref):
    deg = cnt_ref[0] + cnt_ref[1]
    dis = lax.rsqrt(deg)
    h = jnp.dot(x_ref[...], w1_ref[...], preferred_element_type=jnp.float32)
    dis_ref[...] = dis
    g1_ref[0, :N_NODES] = dis[:N_NODES] * h
    g1_ref[0, N_NODES:] = jnp.zeros((NPAD - N_NODES, F), jnp.float32)
    g1_ref[1] = jnp.zeros((NPAD, F), jnp.float32)


def _tc2_body(acc_ref, dis_ref, b1_ref, w2_ref, g2_ref):
    dis = dis_ref[...]
    z1 = jnp.maximum(dis * (acc_ref[0] + acc_ref[1]) + b1_ref[...], 0.0)
    g2_ref[0] = dis * jnp.dot(z1, w2_ref[...],
                              preferred_element_type=jnp.float32)
    g2_ref[1] = jnp.zeros_like(dis)


def _tc3_body(acc_ref, dis_ref, b2_ref, g3_ref):
    dis = dis_ref[...]
    z2 = jnp.maximum(dis * (acc_ref[0] + acc_ref[1]) + b2_ref[...], 0.0)
    g3_ref[0] = dis * z2
    g3_ref[1] = jnp.zeros_like(dis)


def _tc4_body(acc_ref, dis_ref, w3_ref, b3_ref, out_ref):
    p = dis_ref[:N_NODES] * (acc_ref[0, :N_NODES] + acc_ref[1, :N_NODES])
    out_ref[...] = jnp.dot(p, w3_ref[...],
                           preferred_element_type=jnp.float32) + b3_ref[...]


def _f32(*shape):
    return jax.ShapeDtypeStruct(shape, jnp.float32)


def kernel(x, edge_index, W1, b1, W2, b2, W3, b3):
    fin = x.shape[1]
    fout = W3.shape[1]
    e = edge_index.shape[1]
    ept = pl.cdiv(e, NTILES * CHUNK) * CHUNK
    nchunk = ept // CHUNK
    assert nchunk % INNER == 0

    src = edge_index[0].astype(jnp.int32)
    dst = edge_index[1].astype(jnp.int32)
    pad = jnp.full((NTILES * ept - e,), DUMMY, jnp.int32)
    dummy_chunks = jnp.full((NTILES, NBUF, CHUNK), DUMMY, jnp.int32)

    def _slab(idx):
        sl = jnp.concatenate([idx, pad]).reshape(NTILES, nchunk, CHUNK)
        return jnp.concatenate([sl, dummy_chunks], axis=1)

    src3 = _slab(src)
    dst3 = _slab(dst)

    ones_tab = jnp.ones((NPAD, F), jnp.float32)
    init0 = jnp.stack([ones_tab, jnp.zeros((NPAD, F), jnp.float32)])

    agg = _make_agg(nchunk)
    cnt = agg(ones_tab, init0, src3, dst3)
    g1, dis = pl.pallas_call(
        _tc1_body,
        out_shape=(_f32(NCORES, NPAD, F), _f32(NPAD, F)),
    )(x, W1, cnt)
    a1 = agg(g1[0], g1, src3, dst3)
    g2 = pl.pallas_call(
        _tc2_body, out_shape=_f32(NCORES, NPAD, F),
    )(a1, dis, b1.reshape(1, F), W2)
    a2 = agg(g2[0], g2, src3, dst3)
    g3 = pl.pallas_call(
        _tc3_body, out_shape=_f32(NCORES, NPAD, F),
    )(a2, dis, b2.reshape(1, F))
    a3 = agg(g3[0], g3, src3, dst3)
    out = pl.pallas_call(
        _tc4_body, out_shape=_f32(N_NODES, fout),
    )(a3, dis, W3, b3.reshape(1, fout))
    return out


# speedup vs baseline: 1.3408x; 1.3408x over previous
import functools

import jax
import jax.numpy as jnp
from jax import lax
from jax.experimental import pallas as pl
from jax.experimental.pallas import tpu as pltpu
from jax.experimental.pallas import tpu_sc as plsc

N_NODES = 10000
F = 16
NPAD = 10112
ROWS_PER_SUB = NPAD // 16
DUMMY = 10008
NCORES = 2
NSUB = 16
NTILES = NCORES * NSUB
CHUNK = 512

_mesh = plsc.VectorSubcoreMesh(core_axis_name="c", subcore_axis_name="s")


NBUF = 2
INNER = 2


def _agg_body(nchunk, g_hbm, init_hbm, src_hbm, dst_hbm, out_hbm,
              sidx, didx, rows, stage, acc, *sems):
    c = lax.axis_index("c")
    s = lax.axis_index("s")
    t = c * NSUB + s
    sl = pl.ds(s * ROWS_PER_SUB, ROWS_PER_SUB)
    pltpu.sync_copy(init_hbm.at[c, sl], stage)
    pltpu.sync_copy(stage, acc.at[sl])
    pltpu.sync_copy(src_hbm.at[t], sidx)
    pltpu.sync_copy(dst_hbm.at[t], didx)
    plsc.subcore_barrier()

    def _chunk(j, carry):
        pltpu.async_copy(g_hbm.at[sidx.at[j]], rows.at[0], sems[0]).wait()
        pltpu.sync_copy(rows.at[0], acc.at[didx.at[j]], add=True)
        return carry

    lax.fori_loop(0, nchunk, _chunk, 0)
    plsc.subcore_barrier()
    pltpu.sync_copy(acc.at[sl], out_hbm.at[c, sl])


def _make_agg(nchunk):
    return functools.partial(
        pl.kernel,
        mesh=_mesh,
        compiler_params=pltpu.CompilerParams(use_tc_tiling_on_sc=False),
        out_type=jax.ShapeDtypeStruct((NCORES, NPAD, F), jnp.float32),
        scratch_types=[
            pltpu.VMEM((nchunk + NBUF, CHUNK), jnp.int32),
            pltpu.VMEM((nchunk + NBUF, CHUNK), jnp.int32),
            pltpu.VMEM((NBUF, CHUNK, F), jnp.float32),
            pltpu.VMEM((ROWS_PER_SUB, F), jnp.float32),
            pltpu.VMEM_SHARED((NPAD, F), jnp.float32),
        ] + [pltpu.SemaphoreType.DMA] * NBUF,
    )(functools.partial(_agg_body, nchunk))


def _tc1_body(x_ref, w1_ref, cnt_ref, g1_ref, dis_ref):
    deg = cnt_ref[0] + cnt_ref[1]
    dis = lax.rsqrt(deg)
    h = jnp.dot(x_ref[...], w1_ref[...], preferred_element_type=jnp.float32)
    dis_ref[...] = dis
    g1_ref[0, :N_NODES] = dis[:N_NODES] * h
    g1_ref[0, N_NODES:] = jnp.zeros((NPAD - N_NODES, F), jnp.float32)
    g1_ref[1] = jnp.zeros((NPAD, F), jnp.float32)


def _tc2_body(acc_ref, dis_ref, b1_ref, w2_ref, g2_ref):
    dis = dis_ref[...]
    z1 = jnp.maximum(dis * (acc_ref[0] + acc_ref[1]) + b1_ref[...], 0.0)
    g2_ref[0] = dis * jnp.dot(z1, w2_ref[...],
                              preferred_element_type=jnp.float32)
    g2_ref[1] = jnp.zeros_like(dis)


def _tc3_body(acc_ref, dis_ref, b2_ref, g3_ref):
    dis = dis_ref[...]
    z2 = jnp.maximum(dis * (acc_ref[0] + acc_ref[1]) + b2_ref[...], 0.0)
    g3_ref[0] = dis * z2
    g3_ref[1] = jnp.zeros_like(dis)


def _tc4_body(acc_ref, dis_ref, w3_ref, b3_ref, out_ref):
    p = dis_ref[:N_NODES] * (acc_ref[0, :N_NODES] + acc_ref[1, :N_NODES])
    out_ref[...] = jnp.dot(p, w3_ref[...],
                           preferred_element_type=jnp.float32) + b3_ref[...]


def _f32(*shape):
    return jax.ShapeDtypeStruct(shape, jnp.float32)


def kernel(x, edge_index, W1, b1, W2, b2, W3, b3):
    fin = x.shape[1]
    fout = W3.shape[1]
    e = edge_index.shape[1]
    ept = pl.cdiv(e, NTILES * CHUNK) * CHUNK
    nchunk = ept // CHUNK
    assert nchunk % INNER == 0

    src = edge_index[0].astype(jnp.int32)
    dst = edge_index[1].astype(jnp.int32)
    pad = jnp.full((NTILES * ept - e,), DUMMY, jnp.int32)
    dummy_chunks = jnp.full((NTILES, NBUF, CHUNK), DUMMY, jnp.int32)

    def _slab(idx):
        sl = jnp.concatenate([idx, pad]).reshape(NTILES, nchunk, CHUNK)
        return jnp.concatenate([sl, dummy_chunks], axis=1)

    src3 = _slab(src)
    dst3 = _slab(dst)

    ones_tab = jnp.ones((NPAD, F), jnp.float32)
    init0 = jnp.stack([ones_tab, jnp.zeros((NPAD, F), jnp.float32)])

    agg = _make_agg(nchunk)
    cnt = agg(ones_tab, init0, src3, dst3)
    g1, dis = pl.pallas_call(
        _tc1_body,
        out_shape=(_f32(NCORES, NPAD, F), _f32(NPAD, F)),
    )(x, W1, cnt)
    a1 = agg(g1[0], g1, src3, dst3)
    g2 = pl.pallas_call(
        _tc2_body, out_shape=_f32(NCORES, NPAD, F),
    )(a1, dis, b1.reshape(1, F), W2)
    a2 = agg(g2[0], g2, src3, dst3)
    g3 = pl.pallas_call(
        _tc3_body, out_shape=_f32(NCORES, NPAD, F),
    )(a2, dis, b2.reshape(1, F))
    a3 = agg(g3[0], g3, src3, dst3)
    out = pl.pallas_call(
        _tc4_body, out_shape=_f32(N_NODES, fout),
    )(a3, dis, W3, b3.reshape(1, fout))
    return out


# speedup vs baseline: 1.3456x; 1.0035x over previous
import functools

import jax
import jax.numpy as jnp
from jax import lax
from jax.experimental import pallas as pl
from jax.experimental.pallas import tpu as pltpu
from jax.experimental.pallas import tpu_sc as plsc

N_NODES = 10000
F = 16
NPAD = 10112
ROWS_PER_SUB = NPAD // 16
DUMMY = 10008
NCORES = 2
NSUB = 16
NTILES = NCORES * NSUB
CHUNK = 5120

_mesh = plsc.VectorSubcoreMesh(core_axis_name="c", subcore_axis_name="s")


def _agg_body(nchunk, g_hbm, init_hbm, src_hbm, dst_hbm, out_hbm,
              sidx, didx, rows, stage, acc, sem):
    c = lax.axis_index("c")
    s = lax.axis_index("s")
    t = c * NSUB + s
    sl = pl.ds(s * ROWS_PER_SUB, ROWS_PER_SUB)
    pltpu.sync_copy(init_hbm.at[c, sl], stage)
    pltpu.sync_copy(stage, acc.at[sl])
    pltpu.sync_copy(src_hbm.at[t], sidx)
    pltpu.sync_copy(dst_hbm.at[t], didx)
    plsc.subcore_barrier()

    def _chunk(j, carry):
        pltpu.async_copy(g_hbm.at[sidx.at[j]], rows, sem).wait()
        pltpu.sync_copy(rows, acc.at[didx.at[j]], add=True)
        return carry

    lax.fori_loop(0, nchunk, _chunk, 0)
    plsc.subcore_barrier()
    pltpu.sync_copy(acc.at[sl], out_hbm.at[c, sl])


def _make_agg(nchunk):
    return functools.partial(
        pl.kernel,
        mesh=_mesh,
        compiler_params=pltpu.CompilerParams(use_tc_tiling_on_sc=False),
        out_type=jax.ShapeDtypeStruct((NCORES, NPAD, F), jnp.float32),
        scratch_types=[
            pltpu.VMEM((nchunk, CHUNK), jnp.int32),
            pltpu.VMEM((nchunk, CHUNK), jnp.int32),
            pltpu.VMEM((CHUNK, F), jnp.float32),
            pltpu.VMEM((ROWS_PER_SUB, F), jnp.float32),
            pltpu.VMEM_SHARED((NPAD, F), jnp.float32),
            pltpu.SemaphoreType.DMA,
        ],
    )(functools.partial(_agg_body, nchunk))


def _tc1_body(x_ref, w1_ref, cnt_ref, g1_ref, dis_ref):
    deg = cnt_ref[0] + cnt_ref[1]
    dis = lax.rsqrt(deg)
    h = jnp.dot(x_ref[...], w1_ref[...], preferred_element_type=jnp.float32)
    dis_ref[...] = dis
    g1_ref[0, :N_NODES] = dis[:N_NODES] * h
    g1_ref[0, N_NODES:] = jnp.zeros((NPAD - N_NODES, F), jnp.float32)
    g1_ref[1] = jnp.zeros((NPAD, F), jnp.float32)


def _tc2_body(acc_ref, dis_ref, b1_ref, w2_ref, g2_ref):
    dis = dis_ref[...]
    z1 = jnp.maximum(dis * (acc_ref[0] + acc_ref[1]) + b1_ref[...], 0.0)
    g2_ref[0] = dis * jnp.dot(z1, w2_ref[...],
                              preferred_element_type=jnp.float32)
    g2_ref[1] = jnp.zeros_like(dis)


def _tc3_body(acc_ref, dis_ref, b2_ref, g3_ref):
    dis = dis_ref[...]
    z2 = jnp.maximum(dis * (acc_ref[0] + acc_ref[1]) + b2_ref[...], 0.0)
    g3_ref[0] = dis * z2
    g3_ref[1] = jnp.zeros_like(dis)


def _tc4_body(acc_ref, dis_ref, w3_ref, b3_ref, out_ref):
    p = dis_ref[:N_NODES] * (acc_ref[0, :N_NODES] + acc_ref[1, :N_NODES])
    out_ref[...] = jnp.dot(p, w3_ref[...],
                           preferred_element_type=jnp.float32) + b3_ref[...]


def _f32(*shape):
    return jax.ShapeDtypeStruct(shape, jnp.float32)


def kernel(x, edge_index, W1, b1, W2, b2, W3, b3):
    fin = x.shape[1]
    fout = W3.shape[1]
    e = edge_index.shape[1]
    ept = pl.cdiv(e, NTILES * CHUNK) * CHUNK
    nchunk = ept // CHUNK

    src = edge_index[0].astype(jnp.int32)
    dst = edge_index[1].astype(jnp.int32)
    pad = jnp.full((NTILES * ept - e,), DUMMY, jnp.int32)

    def _slab(idx):
        return jnp.concatenate([idx, pad]).reshape(NTILES, nchunk, CHUNK)

    src3 = _slab(src)
    dst3 = _slab(dst)

    ones_tab = jnp.ones((NPAD, F), jnp.float32)
    init0 = jnp.stack([ones_tab, jnp.zeros((NPAD, F), jnp.float32)])

    agg = _make_agg(nchunk)
    cnt = agg(ones_tab, init0, src3, dst3)
    g1, dis = pl.pallas_call(
        _tc1_body,
        out_shape=(_f32(NCORES, NPAD, F), _f32(NPAD, F)),
    )(x, W1, cnt)
    a1 = agg(g1[0], g1, src3, dst3)
    g2 = pl.pallas_call(
        _tc2_body, out_shape=_f32(NCORES, NPAD, F),
    )(a1, dis, b1.reshape(1, F), W2)
    a2 = agg(g2[0], g2, src3, dst3)
    g3 = pl.pallas_call(
        _tc3_body, out_shape=_f32(NCORES, NPAD, F),
    )(a2, dis, b2.reshape(1, F))
    a3 = agg(g3[0], g3, src3, dst3)
    out = pl.pallas_call(
        _tc4_body, out_shape=_f32(N_NODES, fout),
    )(a3, dis, W3, b3.reshape(1, fout))
    return out


# speedup vs baseline: 1.7486x; 1.2995x over previous
import functools

import jax
import jax.numpy as jnp
from jax import lax
from jax.experimental import pallas as pl
from jax.experimental.pallas import tpu as pltpu
from jax.experimental.pallas import tpu_sc as plsc

N_NODES = 10000
F = 16
NPAD = 10112
ROWS_PER_SUB = NPAD // 16
DUMMY = 10008
NCORES = 2
NSUB = 16
NTILES = NCORES * NSUB
CHUNK = 5120

_mesh = plsc.VectorSubcoreMesh(core_axis_name="c", subcore_axis_name="s")


def _agg_body(nchunk, g_hbm, init_hbm, src_hbm, dst_hbm, out_hbm,
              sidx, didx, rows, stage, acc, gtab, sem):
    c = lax.axis_index("c")
    s = lax.axis_index("s")
    t = c * NSUB + s
    sl = pl.ds(s * ROWS_PER_SUB, ROWS_PER_SUB)
    pltpu.sync_copy(init_hbm.at[c, sl], stage)
    pltpu.sync_copy(stage, acc.at[sl])
    pltpu.sync_copy(g_hbm.at[sl], stage)
    pltpu.sync_copy(stage, gtab.at[sl])
    pltpu.sync_copy(src_hbm.at[t], sidx)
    pltpu.sync_copy(dst_hbm.at[t], didx)
    plsc.subcore_barrier()

    def _chunk(j, carry):
        pltpu.async_copy(gtab.at[sidx.at[j]], rows, sem).wait()
        pltpu.sync_copy(rows, acc.at[didx.at[j]], add=True)
        return carry

    lax.fori_loop(0, nchunk, _chunk, 0)
    plsc.subcore_barrier()
    pltpu.sync_copy(acc.at[sl], out_hbm.at[c, sl])


def _make_agg(nchunk):
    return functools.partial(
        pl.kernel,
        mesh=_mesh,
        compiler_params=pltpu.CompilerParams(use_tc_tiling_on_sc=False),
        out_type=jax.ShapeDtypeStruct((NCORES, NPAD, F), jnp.float32),
        scratch_types=[
            pltpu.VMEM((nchunk, CHUNK), jnp.int32),
            pltpu.VMEM((nchunk, CHUNK), jnp.int32),
            pltpu.VMEM((CHUNK, F), jnp.float32),
            pltpu.VMEM((ROWS_PER_SUB, F), jnp.float32),
            pltpu.VMEM_SHARED((NPAD, F), jnp.float32),
            pltpu.VMEM_SHARED((NPAD, F), jnp.float32),
            pltpu.SemaphoreType.DMA,
        ],
    )(functools.partial(_agg_body, nchunk))


def _tc1_body(x_ref, w1_ref, cnt_ref, g1_ref, dis_ref):
    deg = cnt_ref[0] + cnt_ref[1]
    dis = lax.rsqrt(deg)
    h = jnp.dot(x_ref[...], w1_ref[...], preferred_element_type=jnp.float32)
    dis_ref[...] = dis
    g1_ref[0, :N_NODES] = dis[:N_NODES] * h
    g1_ref[0, N_NODES:] = jnp.zeros((NPAD - N_NODES, F), jnp.float32)
    g1_ref[1] = jnp.zeros((NPAD, F), jnp.float32)


def _tc2_body(acc_ref, dis_ref, b1_ref, w2_ref, g2_ref):
    dis = dis_ref[...]
    z1 = jnp.maximum(dis * (acc_ref[0] + acc_ref[1]) + b1_ref[...], 0.0)
    g2_ref[0] = dis * jnp.dot(z1, w2_ref[...],
                              preferred_element_type=jnp.float32)
    g2_ref[1] = jnp.zeros_like(dis)


def _tc3_body(acc_ref, dis_ref, b2_ref, g3_ref):
    dis = dis_ref[...]
    z2 = jnp.maximum(dis * (acc_ref[0] + acc_ref[1]) + b2_ref[...], 0.0)
    g3_ref[0] = dis * z2
    g3_ref[1] = jnp.zeros_like(dis)


def _tc4_body(acc_ref, dis_ref, w3_ref, b3_ref, out_ref):
    p = dis_ref[:N_NODES] * (acc_ref[0, :N_NODES] + acc_ref[1, :N_NODES])
    out_ref[...] = jnp.dot(p, w3_ref[...],
                           preferred_element_type=jnp.float32) + b3_ref[...]


def _f32(*shape):
    return jax.ShapeDtypeStruct(shape, jnp.float32)


def kernel(x, edge_index, W1, b1, W2, b2, W3, b3):
    fin = x.shape[1]
    fout = W3.shape[1]
    e = edge_index.shape[1]
    ept = pl.cdiv(e, NTILES * CHUNK) * CHUNK
    nchunk = ept // CHUNK

    src = edge_index[0].astype(jnp.int32)
    dst = edge_index[1].astype(jnp.int32)
    pad = jnp.full((NTILES * ept - e,), DUMMY, jnp.int32)

    def _slab(idx):
        return jnp.concatenate([idx, pad]).reshape(NTILES, nchunk, CHUNK)

    src3 = _slab(src)
    dst3 = _slab(dst)

    ones_tab = jnp.ones((NPAD, F), jnp.float32)
    init0 = jnp.stack([ones_tab, jnp.zeros((NPAD, F), jnp.float32)])

    agg = _make_agg(nchunk)
    cnt = agg(ones_tab, init0, src3, dst3)
    g1, dis = pl.pallas_call(
        _tc1_body,
        out_shape=(_f32(NCORES, NPAD, F), _f32(NPAD, F)),
    )(x, W1, cnt)
    a1 = agg(g1[0], g1, src3, dst3)
    g2 = pl.pallas_call(
        _tc2_body, out_shape=_f32(NCORES, NPAD, F),
    )(a1, dis, b1.reshape(1, F), W2)
    a2 = agg(g2[0], g2, src3, dst3)
    g3 = pl.pallas_call(
        _tc3_body, out_shape=_f32(NCORES, NPAD, F),
    )(a2, dis, b2.reshape(1, F))
    a3 = agg(g3[0], g3, src3, dst3)
    out = pl.pallas_call(
        _tc4_body, out_shape=_f32(N_NODES, fout),
    )(a3, dis, W3, b3.reshape(1, fout))
    return out


# speedup vs baseline: 2.1990x; 1.2576x over previous
import functools

import jax
import jax.numpy as jnp
from jax import lax
from jax.experimental import pallas as pl
from jax.experimental.pallas import tpu as pltpu
from jax.experimental.pallas import tpu_sc as plsc

N_NODES = 10000
F = 16
NPAD = 10112
ROWS_PER_SUB = NPAD // 16
DUMMY = 10008
NCORES = 2
NSUB = 16
NTILES = NCORES * NSUB

_mesh = plsc.VectorSubcoreMesh(core_axis_name="c", subcore_axis_name="s")


def _agg_body(ept, g_hbm, src_hbm, dst_hbm, out_hbm,
              sidx, didx, rows, stage, acc, gtab, sem):
    c = lax.axis_index("c")
    s = lax.axis_index("s")
    t = c * NSUB + s
    sl = pl.ds(s * ROWS_PER_SUB, ROWS_PER_SUB)
    pltpu.sync_copy(g_hbm.at[sl], stage)
    pltpu.sync_copy(stage, acc.at[sl])
    pltpu.sync_copy(stage, gtab.at[sl])
    pltpu.sync_copy(src_hbm.at[pl.ds(t * ept, ept)], sidx)
    pltpu.sync_copy(dst_hbm.at[pl.ds(t * ept, ept)], didx)
    plsc.subcore_barrier()
    pltpu.async_copy(gtab.at[sidx], rows, sem).wait()
    pltpu.sync_copy(rows, acc.at[didx], add=True)
    plsc.subcore_barrier()
    pltpu.sync_copy(acc.at[sl], out_hbm.at[c, sl])


def _make_agg(ept):
    return functools.partial(
        pl.kernel,
        mesh=_mesh,
        compiler_params=pltpu.CompilerParams(use_tc_tiling_on_sc=False),
        out_type=jax.ShapeDtypeStruct((NCORES, NPAD, F), jnp.float32),
        scratch_types=[
            pltpu.VMEM((ept,), jnp.int32),
            pltpu.VMEM((ept,), jnp.int32),
            pltpu.VMEM((ept, F), jnp.float32),
            pltpu.VMEM((ROWS_PER_SUB, F), jnp.float32),
            pltpu.VMEM_SHARED((NPAD, F), jnp.float32),
            pltpu.VMEM_SHARED((NPAD, F), jnp.float32),
            pltpu.SemaphoreType.DMA,
        ],
    )(functools.partial(_agg_body, ept))


def _tc1a_body(x_ref, w1_ref, h_ref):
    h_ref[...] = jnp.dot(x_ref[...], w1_ref[...],
                         preferred_element_type=jnp.float32)


def _tc1b_body(h_ref, cnt_ref, g1_ref, dis_ref):
    deg = cnt_ref[0] + cnt_ref[1] - 1.0
    dis = lax.rsqrt(deg)
    dis_ref[...] = dis
    g1_ref[:N_NODES] = dis[:N_NODES] * h_ref[...]
    g1_ref[N_NODES:] = jnp.zeros((NPAD - N_NODES, F), jnp.float32)


def _tc2_body(acc_ref, g1_ref, dis_ref, b1_ref, w2_ref, g2_ref):
    dis = dis_ref[...]
    pre = acc_ref[0] + acc_ref[1] - g1_ref[...]
    z1 = jnp.maximum(dis * pre + b1_ref[...], 0.0)
    g2_ref[...] = dis * jnp.dot(z1, w2_ref[...],
                                preferred_element_type=jnp.float32)


def _tc3_body(acc_ref, g2_ref, dis_ref, b2_ref, g3_ref):
    dis = dis_ref[...]
    pre = acc_ref[0] + acc_ref[1] - g2_ref[...]
    z2 = jnp.maximum(dis * pre + b2_ref[...], 0.0)
    g3_ref[...] = dis * z2


def _tc4_body(acc_ref, g3_ref, dis_ref, w3_ref, b3_ref, out_ref):
    pre = acc_ref[0, :N_NODES] + acc_ref[1, :N_NODES] - g3_ref[:N_NODES]
    p = dis_ref[:N_NODES] * pre
    out_ref[...] = jnp.dot(p, w3_ref[...],
                           preferred_element_type=jnp.float32) + b3_ref[...]


def _f32(*shape):
    return jax.ShapeDtypeStruct(shape, jnp.float32)


def kernel(x, edge_index, W1, b1, W2, b2, W3, b3):
    fout = W3.shape[1]
    e = edge_index.shape[1]
    ept = pl.cdiv(e, NTILES * 8) * 8
    epad = NTILES * ept

    src = edge_index[0].astype(jnp.int32)
    dst = edge_index[1].astype(jnp.int32)
    if epad > e:
        pad = jnp.full((epad - e,), DUMMY, jnp.int32)
        src = jnp.concatenate([src, pad])
        dst = jnp.concatenate([dst, pad])

    ones_tab = jnp.ones((NPAD, F), jnp.float32)

    agg = _make_agg(ept)
    cnt = agg(ones_tab, src, dst)
    h1 = pl.pallas_call(
        _tc1a_body, out_shape=_f32(N_NODES, F),
    )(x, W1)
    g1, dis = pl.pallas_call(
        _tc1b_body, out_shape=(_f32(NPAD, F), _f32(NPAD, F)),
    )(h1, cnt)
    a1 = agg(g1, src, dst)
    g2 = pl.pallas_call(
        _tc2_body, out_shape=_f32(NPAD, F),
    )(a1, g1, dis, b1.reshape(1, F), W2)
    a2 = agg(g2, src, dst)
    g3 = pl.pallas_call(
        _tc3_body, out_shape=_f32(NPAD, F),
    )(a2, g2, dis, b2.reshape(1, F))
    a3 = agg(g3, src, dst)
    out = pl.pallas_call(
        _tc4_body, out_shape=_f32(N_NODES, fout),
    )(a3, g3, dis, W3, b3.reshape(1, fout))
    return out


# speedup vs baseline: 2.2531x; 1.0246x over previous
import functools

import jax
import jax.numpy as jnp
from jax import lax
from jax.experimental import pallas as pl
from jax.experimental.pallas import tpu as pltpu
from jax.experimental.pallas import tpu_sc as plsc

N_NODES = 10000
F = 16
NPAD = 10112
ROWS_PER_SUB = NPAD // 16
DUMMY = 10008
NCORES = 2
NSUB = 16
NTILES = NCORES * NSUB

_mesh = plsc.VectorSubcoreMesh(core_axis_name="c", subcore_axis_name="s")


def _agg_body(ept, g_hbm, edge_hbm, out_hbm,
              sidx, didx, rows, stage, acc, gtab, sem):
    c = lax.axis_index("c")
    s = lax.axis_index("s")
    t = c * NSUB + s
    sl = pl.ds(s * ROWS_PER_SUB, ROWS_PER_SUB)
    pltpu.sync_copy(g_hbm.at[sl], stage)
    pltpu.sync_copy(stage, acc.at[sl])
    pltpu.sync_copy(stage, gtab.at[sl])
    pltpu.sync_copy(edge_hbm.at[0, pl.ds(t * ept, ept)], sidx)
    pltpu.sync_copy(edge_hbm.at[1, pl.ds(t * ept, ept)], didx)
    plsc.subcore_barrier()
    pltpu.async_copy(gtab.at[sidx], rows, sem).wait()
    pltpu.sync_copy(rows, acc.at[didx], add=True)
    plsc.subcore_barrier()
    pltpu.sync_copy(acc.at[sl], out_hbm.at[c, sl])


def _make_agg(ept):
    return functools.partial(
        pl.kernel,
        mesh=_mesh,
        compiler_params=pltpu.CompilerParams(use_tc_tiling_on_sc=False),
        out_type=jax.ShapeDtypeStruct((NCORES, NPAD, F), jnp.float32),
        scratch_types=[
            pltpu.VMEM((ept,), jnp.int32),
            pltpu.VMEM((ept,), jnp.int32),
            pltpu.VMEM((ept, F), jnp.float32),
            pltpu.VMEM((ROWS_PER_SUB, F), jnp.float32),
            pltpu.VMEM_SHARED((NPAD, F), jnp.float32),
            pltpu.VMEM_SHARED((NPAD, F), jnp.float32),
            pltpu.SemaphoreType.DMA,
        ],
    )(functools.partial(_agg_body, ept))


def _tc1a_body(x_ref, w1_ref, h_ref):
    h_ref[...] = jnp.dot(x_ref[...], w1_ref[...],
                         preferred_element_type=jnp.float32)


def _tc1b_body(h_ref, cnt_ref, g1_ref, dis_ref):
    deg = cnt_ref[0] + cnt_ref[1] - 1.0
    dis = lax.rsqrt(deg)
    dis_ref[...] = dis
    g1_ref[:N_NODES] = dis[:N_NODES] * h_ref[...]
    g1_ref[N_NODES:] = jnp.zeros((NPAD - N_NODES, F), jnp.float32)


def _tc2_body(acc_ref, g1_ref, dis_ref, b1_ref, w2_ref, g2_ref):
    dis = dis_ref[...]
    pre = acc_ref[0] + acc_ref[1] - g1_ref[...]
    z1 = jnp.maximum(dis * pre + b1_ref[...], 0.0)
    g2_ref[...] = dis * jnp.dot(z1, w2_ref[...],
                                preferred_element_type=jnp.float32)


def _tc3_body(acc_ref, g2_ref, dis_ref, b2_ref, g3_ref):
    dis = dis_ref[...]
    pre = acc_ref[0] + acc_ref[1] - g2_ref[...]
    z2 = jnp.maximum(dis * pre + b2_ref[...], 0.0)
    g3_ref[...] = dis * z2


def _tc4_body(acc_ref, g3_ref, dis_ref, w3_ref, b3_ref, out_ref):
    pre = acc_ref[0, :N_NODES] + acc_ref[1, :N_NODES] - g3_ref[:N_NODES]
    p = dis_ref[:N_NODES] * pre
    out_ref[...] = jnp.dot(p, w3_ref[...],
                           preferred_element_type=jnp.float32) + b3_ref[...]


def _f32(*shape):
    return jax.ShapeDtypeStruct(shape, jnp.float32)


def kernel(x, edge_index, W1, b1, W2, b2, W3, b3):
    fout = W3.shape[1]
    e = edge_index.shape[1]
    ept = pl.cdiv(e, NTILES * 8) * 8
    epad = NTILES * ept

    edges = edge_index.astype(jnp.int32)
    if epad > e:
        edges = jnp.pad(edges, ((0, 0), (0, epad - e)),
                        constant_values=DUMMY)

    ones_tab = jnp.ones((NPAD, F), jnp.float32)

    agg = _make_agg(ept)
    cnt = agg(ones_tab, edges)
    h1 = pl.pallas_call(
        _tc1a_body, out_shape=_f32(N_NODES, F),
    )(x, W1)
    g1, dis = pl.pallas_call(
        _tc1b_body, out_shape=(_f32(NPAD, F), _f32(NPAD, F)),
    )(h1, cnt)
    a1 = agg(g1, edges)
    g2 = pl.pallas_call(
        _tc2_body, out_shape=_f32(NPAD, F),
    )(a1, g1, dis, b1.reshape(1, F), W2)
    a2 = agg(g2, edges)
    g3 = pl.pallas_call(
        _tc3_body, out_shape=_f32(NPAD, F),
    )(a2, g2, dis, b2.reshape(1, F))
    a3 = agg(g3, edges)
    out = pl.pallas_call(
        _tc4_body, out_shape=_f32(N_NODES, fout),
    )(a3, g3, dis, W3, b3.reshape(1, fout))
    return out
